# baseline (speedup 1.0000x reference)
import functools

import jax
import jax.numpy as jnp
from jax import lax
from jax.experimental import pallas as pl
from jax.experimental.pallas import tpu as pltpu
from jax.experimental.pallas import tpu_sc as plsc

VOCAB = 100000
EMBED = 32
BATCH = 1024


_GRP = 128 // EMBED


@functools.lru_cache(maxsize=None)
def _make_sc_gather():
    info = plsc.get_sparse_core_info()
    nc, ns, nl = info.num_cores, info.num_subcores, info.num_lanes
    nw = nc * ns
    b_per_w = BATCH // nw

    mesh = plsc.VectorSubcoreMesh(core_axis_name="c", subcore_axis_name="s")

    @functools.partial(
        pl.kernel,
        mesh=mesh,
        out_type=jax.ShapeDtypeStruct((BATCH, EMBED), jnp.float32),
        compiler_params=pltpu.CompilerParams(
            use_tc_tiling_on_sc=True, needs_layout_passes=False
        ),
        scratch_types=[
            pltpu.VMEM((BATCH,), jnp.int32),
            pltpu.VMEM((b_per_w,), jnp.int32),
            pltpu.VMEM((b_per_w,), jnp.int32),
            pltpu.VMEM((b_per_w, 128), jnp.float32),
            pltpu.VMEM((b_per_w, EMBED), jnp.float32),
            pltpu.SemaphoreType.DMA,
        ],
    )
    def sc_gather(table4_hbm, idx_hbm, out_hbm, idx_v, grp_v, off_v, groups_v, rows_v, sem):
        wid = lax.axis_index("s") * nc + lax.axis_index("c")
        base = wid * b_per_w
        pltpu.sync_copy(idx_hbm, idx_v)
        for h in range(b_per_w // nl):
            iv = idx_v[pl.ds(base + h * nl, nl)]
            grp_v[pl.ds(h * nl, nl)] = jax.lax.shift_right_logical(iv, 2)
            off_v[pl.ds(h * nl, nl)] = (iv & 3) * EMBED
        pltpu.async_copy(table4_hbm.at[grp_v], groups_v, sem).wait()
        lanes = jax.lax.iota(jnp.int32, nl)
        for h in range(b_per_w // nl):
            jrow = h * nl + lanes
            off = off_v[pl.ds(h * nl, nl)]
            for k in range(EMBED):
                v = plsc.load_gather(groups_v, [jrow, off + k])
                plsc.store_scatter(rows_v, [jrow, jnp.full((nl,), k, jnp.int32)], v)
        pltpu.sync_copy(rows_v, out_hbm.at[pl.ds(base, b_per_w)])

    return sc_gather


_NBUF = 8
_TV = 1024
_WIDTH = _NBUF * _TV
_GRID = pl.cdiv(VOCAB, _WIDTH)
_TAIL = [
    max(0, min(_TV, VOCAB - (_GRID - 1) * _WIDTH - t * _TV)) for t in range(_NBUF)
]


def _mm_body(ctx_ref, w_ref, b_ref, out_ref, scratch, sems):
    i = pl.program_id(0)
    last = _GRID - 1
    ctx = jnp.concatenate(
        [ctx_ref[...], jnp.ones((BATCH, 1), jnp.float32)], axis=1
    )
    w_aug = jnp.concatenate([w_ref[...], b_ref[...]], axis=0)

    def mk(t, step, rows):
        return pltpu.make_async_copy(
            scratch.at[t, :rows, :],
            out_ref.at[pl.ds(step * _WIDTH + t * _TV, rows), :],
            sems.at[t],
        )

    for t in range(_NBUF):

        @pl.when(i > 0)
        def _wait_prev():
            mk(t, i - 1, _TV).wait()

        def compute():
            scratch[t] = lax.dot_general(
                w_aug[:, t * _TV : (t + 1) * _TV],
                ctx,
                (((0,), (1,)), ((), ())),
                preferred_element_type=jnp.float32,
            )

        if _TAIL[t] > 0:
            compute()

            @pl.when(i < last)
            def _start_full():
                mk(t, i, _TV).start()

            @pl.when(i == last)
            def _start_tail():
                mk(t, i, _TAIL[t]).start()

        else:
            @pl.when(i < last)
            def _compute_and_start():
                compute()
                mk(t, i, _TV).start()

    @pl.when(i == last)
    def _drain():
        for t in range(_NBUF):
            if _TAIL[t] > 0:
                mk(t, i, _TAIL[t]).wait()


def _decoder_t(context, W, bcol):
    return pl.pallas_call(
        _mm_body,
        grid=(_GRID,),
        in_specs=[
            pl.BlockSpec((BATCH, EMBED), lambda i: (0, 0)),
            pl.BlockSpec((EMBED, _WIDTH), lambda i: (0, i)),
            pl.BlockSpec((1, _WIDTH), lambda i: (0, i)),
        ],
        out_specs=pl.BlockSpec(memory_space=pl.ANY),
        out_shape=jax.ShapeDtypeStruct((VOCAB, BATCH), jnp.float32),
        scratch_shapes=[
            pltpu.VMEM((_NBUF, _TV, BATCH), jnp.float32),
            pltpu.SemaphoreType.DMA((_NBUF,)),
        ],
        compiler_params=pltpu.CompilerParams(vmem_limit_bytes=100 * 1024 * 1024),
    )(context, W, bcol)


@jax.jit
def kernel(x, table, W, b):
    context = _make_sc_gather()(
        table.reshape(VOCAB // _GRP, 128), x.astype(jnp.int32)
    )
    logits_t = _decoder_t(context, W, b.reshape(1, VOCAB))
    return logits_t.T


# speedup vs baseline: 1.0097x; 1.0097x over previous
import functools

import jax
import jax.numpy as jnp
from jax import lax
from jax.experimental import pallas as pl
from jax.experimental.pallas import tpu as pltpu
from jax.experimental.pallas import tpu_sc as plsc

VOCAB = 100000
EMBED = 32
BATCH = 1024


@functools.lru_cache(maxsize=None)
def _make_sc_gather():
    info = plsc.get_sparse_core_info()
    nc, ns = info.num_cores, info.num_subcores
    nw = nc * ns
    b_per_w = BATCH // nw

    mesh = plsc.VectorSubcoreMesh(core_axis_name="c", subcore_axis_name="s")

    @functools.partial(
        pl.kernel,
        mesh=mesh,
        out_type=jax.ShapeDtypeStruct((BATCH, EMBED), jnp.float32),
        compiler_params=pltpu.CompilerParams(use_tc_tiling_on_sc=False),
        scratch_types=[
            pltpu.VMEM((b_per_w,), jnp.int32),
            pltpu.VMEM((b_per_w, EMBED), jnp.float32),
            pltpu.SemaphoreType.DMA,
        ],
    )
    def sc_gather(table_hbm, idx_hbm, out_hbm, idx_v, rows_v, sem):
        wid = lax.axis_index("s") * nc + lax.axis_index("c")
        base = wid * b_per_w
        pltpu.sync_copy(idx_hbm.at[pl.ds(base, b_per_w)], idx_v)
        pltpu.async_copy(table_hbm.at[idx_v], rows_v, sem).wait()
        pltpu.sync_copy(rows_v, out_hbm.at[pl.ds(base, b_per_w)])

    return sc_gather


_NBUF = 8
_TV = 1024
_WIDTH = _NBUF * _TV
_GRID = pl.cdiv(VOCAB, _WIDTH)
_TAIL = [
    max(0, min(_TV, VOCAB - (_GRID - 1) * _WIDTH - t * _TV)) for t in range(_NBUF)
]


def _mm_body(ctx_ref, w_ref, b_ref, out_ref, scratch, sems):
    i = pl.program_id(0)
    last = _GRID - 1
    ctx = jnp.concatenate(
        [ctx_ref[...], jnp.ones((BATCH, 1), jnp.float32)], axis=1
    )
    w_aug = jnp.concatenate([w_ref[...], b_ref[...]], axis=0)

    def mk(t, step, rows):
        return pltpu.make_async_copy(
            scratch.at[t, :rows, :],
            out_ref.at[pl.ds(step * _WIDTH + t * _TV, rows), :],
            sems.at[t],
        )

    for t in range(_NBUF):

        @pl.when(i > 0)
        def _wait_prev():
            mk(t, i - 1, _TV).wait()

        def compute():
            scratch[t] = lax.dot_general(
                w_aug[:, t * _TV : (t + 1) * _TV],
                ctx,
                (((0,), (1,)), ((), ())),
                preferred_element_type=jnp.float32,
            )

        if _TAIL[t] > 0:
            compute()

            @pl.when(i < last)
            def _start_full():
                mk(t, i, _TV).start()

            @pl.when(i == last)
            def _start_tail():
                mk(t, i, _TAIL[t]).start()

        else:
            @pl.when(i < last)
            def _compute_and_start():
                compute()
                mk(t, i, _TV).start()

    @pl.when(i == last)
    def _drain():
        for t in range(_NBUF):
            if _TAIL[t] > 0:
                mk(t, i, _TAIL[t]).wait()


def _decoder_t(context, W, bcol):
    return pl.pallas_call(
        _mm_body,
        grid=(_GRID,),
        in_specs=[
            pl.BlockSpec((BATCH, EMBED), lambda i: (0, 0)),
            pl.BlockSpec((EMBED, _WIDTH), lambda i: (0, i)),
            pl.BlockSpec((1, _WIDTH), lambda i: (0, i)),
        ],
        out_specs=pl.BlockSpec(memory_space=pl.ANY),
        out_shape=jax.ShapeDtypeStruct((VOCAB, BATCH), jnp.float32),
        scratch_shapes=[
            pltpu.VMEM((_NBUF, _TV, BATCH), jnp.float32),
            pltpu.SemaphoreType.DMA((_NBUF,)),
        ],
        compiler_params=pltpu.CompilerParams(vmem_limit_bytes=100 * 1024 * 1024),
    )(context, W, bcol)


@jax.jit
def kernel(x, table, W, b):
    context = _make_sc_gather()(table, x.astype(jnp.int32))
    logits_t = _decoder_t(context, W, b.reshape(1, VOCAB))
    return logits_t.T
